# baseline (speedup 1.0000x reference)
import jax
import jax.numpy as jnp
from jax import lax
from jax.experimental import pallas as pl
from jax.experimental.pallas import tpu as pltpu
from jax.experimental.pallas import tpu_sc as plsc

_BN = 16


def _sc_lookup_body(table_hbm, rows_hbm, rows_v, sem):
    first = (lax.axis_index("c") == 0) & (lax.axis_index("s") == 0)

    @pl.when(first)
    def _():
        S = rows_v.shape[0]
        pltpu.async_copy(table_hbm.at[pl.ds(0, S)], rows_v, sem).wait()
        pltpu.sync_copy(rows_v, rows_hbm)


def _tc_broadcast_body(rows_ref, out_ref):
    out_ref[...] = jnp.broadcast_to(rows_ref[...][None], out_ref.shape)


def kernel(x, pos_table):
    N, S = x.shape
    D = pos_table.shape[1]

    mesh = plsc.VectorSubcoreMesh(core_axis_name="c", subcore_axis_name="s")
    sc_lookup = pl.kernel(
        _sc_lookup_body,
        out_type=jax.ShapeDtypeStruct((S, D), jnp.float32),
        mesh=mesh,
        scratch_types=[
            pltpu.VMEM((S, D), jnp.float32),
            pltpu.SemaphoreType.DMA,
        ],
    )
    rows = sc_lookup(pos_table)

    bn = _BN
    while N % bn:
        bn //= 2
    return pl.pallas_call(
        _tc_broadcast_body,
        grid=(N // bn,),
        in_specs=[pl.BlockSpec((S, D), lambda i: (0, 0))],
        out_specs=pl.BlockSpec((bn, S, D), lambda i: (i, 0, 0)),
        out_shape=jax.ShapeDtypeStruct((N, S, D), jnp.float32),
    )(rows)


# speedup vs baseline: 5.4295x; 5.4295x over previous
import jax
import jax.numpy as jnp
from jax import lax
from jax.experimental import pallas as pl
from jax.experimental.pallas import tpu as pltpu
from jax.experimental.pallas import tpu_sc as plsc

_BS = 8


def _sc_lookup_body(table_hbm, rows_hbm, rows_v, sem):
    first = (lax.axis_index("c") == 0) & (lax.axis_index("s") == 0)

    @pl.when(first)
    def _():
        S = rows_v.shape[0]
        pltpu.async_copy(table_hbm.at[pl.ds(0, S)], rows_v, sem).wait()
        pltpu.sync_copy(rows_v, rows_hbm)


def _tc_splat_body(rows_ref, out_ref):
    out_ref[...] = jnp.broadcast_to(
        rows_ref[...][:, :, None], out_ref.shape
    )


def kernel(x, pos_table):
    N, S = x.shape
    D = pos_table.shape[1]

    mesh = plsc.VectorSubcoreMesh(core_axis_name="c", subcore_axis_name="s")
    sc_lookup = pl.kernel(
        _sc_lookup_body,
        out_type=jax.ShapeDtypeStruct((S, D), jnp.float32),
        mesh=mesh,
        scratch_types=[
            pltpu.VMEM((S, D), jnp.float32),
            pltpu.SemaphoreType.DMA,
        ],
    )
    rows = sc_lookup(pos_table)

    bs = _BS
    while S % bs:
        bs //= 2
    out_t = pl.pallas_call(
        _tc_splat_body,
        grid=(S // bs,),
        in_specs=[pl.BlockSpec((bs, D), lambda i: (i, 0))],
        out_specs=pl.BlockSpec((bs, D, N), lambda i: (i, 0, 0)),
        out_shape=jax.ShapeDtypeStruct((S, D, N), jnp.float32),
    )(rows)
    return jnp.transpose(out_t, (2, 0, 1))
